# baseline (speedup 1.0000x reference)
import jax
import jax.numpy as jnp
from jax import lax
from jax.experimental import pallas as pl
from jax.experimental.pallas import tpu as pltpu
from jax.experimental.pallas import tpu_sc as plsc

NUM_ENTITIES = 100000
NUM_RELATIONS = 1000
EMBED_DIM = 64
BATCH = 16384

NC = 2
NS = 16
L = 16
NW = NC * NS
B_PER_W = BATCH // NW
N_GROUPS = B_PER_W // L
CHUNKS = EMBED_DIM // L


def _vsqrt(x):
    i = plsc.bitcast(x, jnp.int32)
    y = plsc.bitcast(jnp.int32(0x5F3759DF) - (i >> 1), jnp.float32)
    for _ in range(3):
        y = y * (1.5 - 0.5 * x * y * y)
    return x * y


def _body(heads_hbm, relations_hbm, tails_hbm, ent_hbm, rel_hbm, out_hbm,
          idx_h, idx_r, idx_t, h_rows, r_rows, t_rows, out_v,
          sem0, sem1):
    sems = (sem0, sem1)
    wid = lax.axis_index("s") * NC + lax.axis_index("c")
    base = wid * B_PER_W

    pltpu.sync_copy(heads_hbm.at[pl.ds(base, B_PER_W)], idx_h)
    pltpu.sync_copy(relations_hbm.at[pl.ds(base, B_PER_W)], idx_r)
    pltpu.sync_copy(tails_hbm.at[pl.ds(base, B_PER_W)], idx_t)

    def fire(g, slot):
        gb = g * L
        s = pl.ds(gb, L)
        vh = idx_h[s]
        vr = idx_r[s]
        vt = idx_t[s]
        sm = sems[slot]
        for j in range(L):
            pltpu.async_copy(ent_hbm.at[vh[j]], h_rows.at[slot, j], sm)
            pltpu.async_copy(rel_hbm.at[vr[j]], r_rows.at[slot, j], sm)
            pltpu.async_copy(ent_hbm.at[vt[j]], t_rows.at[slot, j], sm)

    def drain(slot):
        sm = sems[slot]
        pltpu.make_async_copy(
            ent_hbm.at[pl.ds(0, L)], h_rows.at[0], sm).wait()
        pltpu.make_async_copy(
            rel_hbm.at[pl.ds(0, L)], r_rows.at[0], sm).wait()
        pltpu.make_async_copy(
            ent_hbm.at[pl.ds(0, L)], t_rows.at[0], sm).wait()

    iota = lax.iota(jnp.int32, L)

    def compute(g, slot):
        gv = jnp.zeros((L,), jnp.float32)
        for j in range(L):
            acc = jnp.zeros((L,), jnp.float32)
            for k in range(CHUNKS):
                h = h_rows[slot, j, pl.ds(k * L, L)]
                r = r_rows[slot, j, pl.ds(k * L, L)]
                t = t_rows[slot, j, pl.ds(k * L, L)]
                df = h + r - t + 1e-6
                acc = acc + df * df
            gv = jnp.where(iota == j, jnp.sum(acc), gv)
        out_v[pl.ds(g * L, L)] = _vsqrt(gv)

    NPH = 2
    for ph in range(NPH):
        fire(ph, ph)

    def step(q, _):
        for ph in range(NPH):
            g = q * NPH + ph
            drain(ph)
            compute(g, ph)
            fire(g + NPH, ph)
        return 0

    lax.fori_loop(0, N_GROUPS // NPH - 1, step, 0)
    for ph in range(NPH):
        g = N_GROUPS - NPH + ph
        drain(ph)
        compute(g, ph)

    pltpu.sync_copy(out_v, out_hbm.at[pl.ds(base, B_PER_W)])


def _transpose_body(x_ref, o_ref):
    o_ref[...] = x_ref[...].T


def _tc_transpose(ent_t):
    blk = 2048
    return pl.pallas_call(
        _transpose_body,
        out_shape=jax.ShapeDtypeStruct((NUM_ENTITIES, EMBED_DIM), jnp.float32),
        grid=((NUM_ENTITIES + blk - 1) // blk,),
        in_specs=[pl.BlockSpec((EMBED_DIM, blk), lambda i: (0, i))],
        out_specs=pl.BlockSpec((blk, EMBED_DIM), lambda i: (i, 0)),
    )(ent_t)


@jax.jit
def _transe(heads, relations, tails, entity_emb, relation_emb):
    ent_rows = _tc_transpose(entity_emb.T)
    mesh = plsc.VectorSubcoreMesh(
        core_axis_name="c", subcore_axis_name="s", num_cores=NC,
        num_subcores=NS)
    return pl.kernel(
        _body,
        out_type=jax.ShapeDtypeStruct((BATCH,), jnp.float32),
        mesh=mesh,
        scratch_types=[
            pltpu.VMEM((B_PER_W,), jnp.int32),
            pltpu.VMEM((B_PER_W,), jnp.int32),
            pltpu.VMEM((B_PER_W,), jnp.int32),
            pltpu.VMEM((2, L, EMBED_DIM), jnp.float32),
            pltpu.VMEM((2, L, EMBED_DIM), jnp.float32),
            pltpu.VMEM((2, L, EMBED_DIM), jnp.float32),
            pltpu.VMEM((B_PER_W,), jnp.float32),
            pltpu.SemaphoreType.DMA,
            pltpu.SemaphoreType.DMA,
        ],
        compiler_params=pltpu.CompilerParams(
            needs_layout_passes=False, use_tc_tiling_on_sc=True),
    )(heads, relations, tails, ent_rows, relation_emb)


def kernel(heads, relations, tails, entity_emb, relation_emb):
    return _transe(heads, relations, tails, entity_emb, relation_emb)


# speedup vs baseline: 1.1652x; 1.1652x over previous
import jax
import jax.numpy as jnp
from jax import lax
from jax.experimental import pallas as pl
from jax.experimental.pallas import tpu as pltpu
from jax.experimental.pallas import tpu_sc as plsc

NUM_ENTITIES = 100000
NUM_RELATIONS = 1000
EMBED_DIM = 64
BATCH = 16384

NC = 2
NS = 16
L = 16
NW = NC * NS
B_PER_W = BATCH // NW
N_GROUPS = B_PER_W // L
CHUNKS = EMBED_DIM // L


def _vsqrt(x):
    i = plsc.bitcast(x, jnp.int32)
    y = plsc.bitcast(jnp.int32(0x5F3759DF) - (i >> 1), jnp.float32)
    for _ in range(3):
        y = y * (1.5 - 0.5 * x * y * y)
    return x * y


def _body(heads_hbm, relations_hbm, tails_hbm, ent_hbm, rel_hbm, out_hbm,
          idx_h, idx_r, idx_t, h0, h1, r0, r1, t0, t1, out_v,
          sem0, sem1):
    sems = (sem0, sem1)
    hb = (h0, h1)
    rb = (r0, r1)
    tb = (t0, t1)
    wid = lax.axis_index("s") * NC + lax.axis_index("c")
    base = wid * B_PER_W

    pltpu.sync_copy(heads_hbm.at[pl.ds(base, B_PER_W)], idx_h)
    pltpu.sync_copy(relations_hbm.at[pl.ds(base, B_PER_W)], idx_r)
    pltpu.sync_copy(tails_hbm.at[pl.ds(base, B_PER_W)], idx_t)

    def fire(g, slot):
        gb = g * L
        s = pl.ds(gb, L)
        vh = idx_h[s]
        vr = idx_r[s]
        vt = idx_t[s]
        sm = sems[slot]
        for j in range(L):
            pltpu.async_copy(ent_hbm.at[vh[j]], hb[slot].at[j], sm)
            pltpu.async_copy(rel_hbm.at[vr[j]], rb[slot].at[j], sm)
            pltpu.async_copy(ent_hbm.at[vt[j]], tb[slot].at[j], sm)

    def drain(slot):
        sm = sems[slot]
        pltpu.make_async_copy(
            ent_hbm.at[pl.ds(0, L)], hb[slot], sm).wait()
        pltpu.make_async_copy(
            rel_hbm.at[pl.ds(0, L)], rb[slot], sm).wait()
        pltpu.make_async_copy(
            ent_hbm.at[pl.ds(0, L)], tb[slot], sm).wait()

    iota = lax.iota(jnp.int32, L)

    def compute(g, slot):
        gv = jnp.zeros((L,), jnp.float32)
        for j in range(L):
            acc = jnp.zeros((L,), jnp.float32)
            for k in range(CHUNKS):
                h = hb[slot][j, pl.ds(k * L, L)]
                r = rb[slot][j, pl.ds(k * L, L)]
                t = tb[slot][j, pl.ds(k * L, L)]
                df = h + r - t + 1e-6
                acc = acc + df * df
            gv = jnp.where(iota == j, jnp.sum(acc), gv)
        out_v[pl.ds(g * L, L)] = _vsqrt(gv)

    NPH = 2
    for ph in range(NPH):
        fire(ph, ph)

    def step(q, _):
        for ph in range(NPH):
            g = q * NPH + ph
            drain(ph)
            compute(g, ph)
            fire(g + NPH, ph)
        return 0

    lax.fori_loop(0, N_GROUPS // NPH - 1, step, 0)
    for ph in range(NPH):
        g = N_GROUPS - NPH + ph
        drain(ph)
        compute(g, ph)

    pltpu.sync_copy(out_v, out_hbm.at[pl.ds(base, B_PER_W)])


def _transpose_body(x_ref, o_ref):
    o_ref[...] = x_ref[...].T


def _tc_transpose(ent_t):
    blk = 2048
    return pl.pallas_call(
        _transpose_body,
        out_shape=jax.ShapeDtypeStruct((NUM_ENTITIES, EMBED_DIM), jnp.float32),
        grid=((NUM_ENTITIES + blk - 1) // blk,),
        in_specs=[pl.BlockSpec((EMBED_DIM, blk), lambda i: (0, i))],
        out_specs=pl.BlockSpec((blk, EMBED_DIM), lambda i: (i, 0)),
    )(ent_t)


@jax.jit
def _transe(heads, relations, tails, entity_emb, relation_emb):
    mesh = plsc.VectorSubcoreMesh(
        core_axis_name="c", subcore_axis_name="s", num_cores=NC,
        num_subcores=NS)
    return pl.kernel(
        _body,
        out_type=jax.ShapeDtypeStruct((BATCH,), jnp.float32),
        mesh=mesh,
        scratch_types=[
            pltpu.VMEM((B_PER_W,), jnp.int32),
            pltpu.VMEM((B_PER_W,), jnp.int32),
            pltpu.VMEM((B_PER_W,), jnp.int32),
            pltpu.VMEM((L, EMBED_DIM), jnp.float32),
            pltpu.VMEM((L, EMBED_DIM), jnp.float32),
            pltpu.VMEM((L, EMBED_DIM), jnp.float32),
            pltpu.VMEM((L, EMBED_DIM), jnp.float32),
            pltpu.VMEM((L, EMBED_DIM), jnp.float32),
            pltpu.VMEM((L, EMBED_DIM), jnp.float32),
            pltpu.VMEM((B_PER_W,), jnp.float32),
            pltpu.SemaphoreType.DMA,
            pltpu.SemaphoreType.DMA,
        ],
        compiler_params=pltpu.CompilerParams(
            needs_layout_passes=False, use_tc_tiling_on_sc=True),
    )(heads, relations, tails, entity_emb, relation_emb)


def kernel(heads, relations, tails, entity_emb, relation_emb):
    return _transe(heads, relations, tails, entity_emb, relation_emb)
